# Initial kernel scaffold; baseline (speedup 1.0000x reference)
import jax
import jax.numpy as jnp
from jax.experimental import pallas as pl


def kernel(raw, edge_index, edge_attr, batch, params):
    raise NotImplementedError("write your pallas kernel here")


# speedup vs baseline: 6.5708x; 6.5708x over previous
import dataclasses
import functools

import jax
import jax.numpy as jnp
from jax import lax
from jax.experimental import pallas as pl
from jax.experimental.pallas import tpu as pltpu
from jax.experimental.pallas import tpu_sc as plsc

N = 10000
E = 320000
F_IN = 128
H = 128
DE = 16
OUT = 128
G = 512
NUM_TIMESTEPS = 2

NP = 10240
GP = 640
NB = 512
NCORES = 2
NSUB = 16
NW = NCORES * NSUB
EPW = E // NW
KE0 = 80
KE1 = 80
NPW = NP // NW
F32 = jnp.float32

_MESH = plsc.VectorSubcoreMesh(core_axis_name="c", subcore_axis_name="s",
                               num_cores=NCORES, num_subcores=NSUB)

_SC_PARAMS = pltpu.CompilerParams()
if "needs_layout_passes" in pltpu.CompilerParams.__dataclass_fields__:
    _SC_PARAMS = dataclasses.replace(_SC_PARAMS, needs_layout_passes=False)


def _lrelu(v):
    return jnp.where(v > 0, v, 0.01 * v)


def _elu(v):
    return jnp.where(v > 0, v, jnp.exp(jnp.minimum(v, 0.0)) - 1.0)


def _dot_t(a, b):
    return lax.dot_general(a, b, (((1,), (1,)), ((), ())),
                           preferred_element_type=F32)


def _t1a_body(raw_ref, w1_ref, b1_ref, wn1_ref, bn_ref, wa0i_ref, ba0_ref,
              x1_ref, z_ref, ad_ref):
    raw = raw_ref[...]
    x1 = _lrelu(_dot_t(raw, w1_ref[...]) + b1_ref[...])
    x1_ref[...] = x1
    z_ref[...] = _dot_t(raw, wn1_ref[...]) + bn_ref[...]
    ad_ref[...] = jnp.dot(x1, wa0i_ref[...], preferred_element_type=F32) \
        + ba0_ref[...]


def _t1b_body(ea_ref, wn2_ref, c_ref):
    c_ref[...] = _dot_t(ea_ref[...], wn2_ref[...])


def _gru(xg, hg, wih, whh, bih, bhh):
    gi = _dot_t(xg, wih) + bih
    gh = _dot_t(hg, whh) + bhh
    r = jax.nn.sigmoid(gi[:, :H] + gh[:, :H])
    z = jax.nn.sigmoid(gi[:, H:2 * H] + gh[:, H:2 * H])
    n = jnp.tanh(gi[:, 2 * H:] + r * gh[:, 2 * H:])
    return (1.0 - z) * n + z * hg


def _node_update_body(acc_ref, s_ref, xp_ref, wat_ref, bat_ref,
                      wih_ref, whh_ref, bih_ref, bhh_ref,
                      wi_ref, wj_ref, bia_ref,
                      x_ref, ai_ref, aj_ref):
    ssum = s_ref[0] + s_ref[1]
    inv = 1.0 / (ssum + 1e-16)
    sa = ssum * inv
    hpre = (acc_ref[0] + acc_ref[1]) * inv
    h = _elu(_dot_t(hpre, wat_ref[...]) + bat_ref[...] * sa)
    xnew = jax.nn.relu(_gru(h, xp_ref[...], wih_ref[...], whh_ref[...],
                            bih_ref[...], bhh_ref[...]))
    x_ref[...] = xnew
    ai_ref[...] = jnp.dot(xnew, wi_ref[...], preferred_element_type=F32) \
        + bia_ref[...]
    aj_ref[...] = jnp.dot(xnew, wj_ref[...], preferred_element_type=F32)


def _t4_body(accr_ref, wami_ref, out_ref, ob_ref):
    out0 = jax.nn.relu(accr_ref[0] + accr_ref[1])
    out_ref[...] = out0
    ob_ref[...] = jnp.dot(out0, wami_ref[...], preferred_element_type=F32)


def _readout_update_body(acc_ref, s_ref, op_ref, wat_ref, bat_ref,
                         wih_ref, whh_ref, bih_ref, bhh_ref, wami_ref,
                         out_ref, ob_ref):
    ssum = s_ref[0] + s_ref[1]
    inv = 1.0 / (ssum + 1e-16)
    sa = ssum * inv
    hpre = (acc_ref[0] + acc_ref[1]) * inv
    h = _elu(_dot_t(hpre, wat_ref[...]) + bat_ref[...] * sa)
    onew = jax.nn.relu(_gru(h, op_ref[...], wih_ref[...], whh_ref[...],
                            bih_ref[...], bhh_ref[...]))
    out_ref[...] = onew
    ob_ref[...] = jnp.dot(onew, wami_ref[...], preferred_element_type=F32)


def _final_body(acc_ref, s_ref, op_ref, wat_ref, bat_ref,
                wih_ref, whh_ref, bih_ref, bhh_ref, w2_ref, b2_ref,
                out_ref):
    ssum = s_ref[0] + s_ref[1]
    inv = 1.0 / (ssum + 1e-16)
    sa = ssum * inv
    hpre = (acc_ref[0] + acc_ref[1]) * inv
    h = _elu(_dot_t(hpre, wat_ref[...]) + bat_ref[...] * sa)
    onew = jax.nn.relu(_gru(h, op_ref[...], wih_ref[...], whh_ref[...],
                            bih_ref[...], bhh_ref[...]))
    out_ref[...] = _dot_t(onew, w2_ref[...]) + b2_ref[...]


def _full(shape):
    return pl.BlockSpec(shape, lambda i: tuple(0 for _ in shape))


def _zero_vec(ref, n):
    @pl.loop(0, n, step=16)
    def _(i):
        ref[pl.ds(i, 16)] = jnp.zeros((16,), F32)


def _zero_rows(ref, rows):
    @pl.loop(0, rows)
    def _(i):
        for k in range(8):
            ref[i, pl.ds(k * 16, 16)] = jnp.zeros((16,), F32)


def _scale_rows(rows_ref, ebuf, nrows):
    @pl.loop(0, nrows)
    def _(i):
        eb = plsc.load_gather(ebuf, [jnp.full((16,), i, jnp.int32)])
        for k in range(8):
            rows_ref[i, pl.ds(k * 16, 16)] = rows_ref[i, pl.ds(k * 16, 16)] * eb


def _l0_sc(z_hbm, c_hbm, src_hbm, dst_hbm, ad_hbm, wa_hbm,
           acc_out, s_out,
           src_v, dst_v, zrows, crows, ebuf, adtab, watab,
           zbuf, zvec, accsh, s_sh, sem):
    c = lax.axis_index("c")
    sid = lax.axis_index("s")
    wid = c * NSUB + sid
    _zero_rows(zbuf, 80)
    for j in range(8):
        pltpu.sync_copy(zbuf, accsh.at[pl.ds(sid * 640 + j * 80, 80)])
    _zero_vec(zvec, NP // NSUB)
    pltpu.sync_copy(zvec, s_sh.at[pl.ds(sid * (NP // NSUB), NP // NSUB)])
    pltpu.sync_copy(ad_hbm, adtab)
    pltpu.sync_copy(wa_hbm, watab)
    plsc.subcore_barrier()

    base0 = wid * EPW

    @pl.loop(0, EPW // KE0)
    def _(b):
        base = base0 + b * KE0
        pltpu.sync_copy(src_hbm.at[pl.ds(base, KE0)], src_v)
        pltpu.sync_copy(dst_hbm.at[pl.ds(base, KE0)], dst_v)
        pltpu.async_copy(z_hbm.at[src_v], zrows, sem).wait()
        pltpu.sync_copy(c_hbm.at[pl.ds(base, KE0)], crows)

        @pl.loop(0, KE0 // 16)
        def _(g):
            dst16 = dst_v[pl.ds(g * 16, 16)]
            ad16 = plsc.load_gather(adtab, [dst16])

            def edge_dot(e2, qv):
                i = g * 16 + e2
                q = jnp.zeros((16,), F32)
                for k in range(8):
                    xs = zrows[i, pl.ds(k * 16, 16)] \
                        + crows[i, pl.ds(k * 16, 16)]
                    xs = jnp.where(xs > 0, xs, 0.01 * xs)
                    crows[i, pl.ds(k * 16, 16)] = xs
                    q = q + xs * watab[pl.ds(k * 16, 16)]
                return jnp.where(lax.iota(jnp.int32, 16) == e2,
                                 jnp.sum(q), qv)

            qv = lax.fori_loop(0, 16, edge_dot, jnp.zeros((16,), F32))
            a16 = _lrelu(ad16 + qv)
            e16 = jnp.exp(jnp.minimum(a16, 60.0))
            ebuf[pl.ds(g * 16, 16)] = e16

        _scale_rows(crows, ebuf, KE0)
        pltpu.sync_copy(crows, accsh.at[dst_v], add=True)
        pltpu.sync_copy(ebuf, s_sh.at[dst_v], add=True)

    plsc.subcore_barrier()
    for j in range(8):
        pltpu.sync_copy(accsh.at[pl.ds(sid * 640 + j * 80, 80)], zbuf)
        pltpu.sync_copy(zbuf, acc_out.at[c, pl.ds(sid * 640 + j * 80, 80)])
    pltpu.sync_copy(s_sh.at[pl.ds(sid * 640, 640)], zvec)
    pltpu.sync_copy(zvec, s_out.at[pl.ds(c * NP + sid * 640, 640)])


def _l1_sc(x_hbm, src_hbm, dst_hbm, ai_hbm, aj_hbm,
           acc_out, s_out,
           src_v, dst_v, rows, ebuf, aitab, ajtab,
           zbuf, zvec, accsh, s_sh, sem):
    c = lax.axis_index("c")
    sid = lax.axis_index("s")
    wid = c * NSUB + sid
    _zero_rows(zbuf, 80)
    for j in range(8):
        pltpu.sync_copy(zbuf, accsh.at[pl.ds(sid * 640 + j * 80, 80)])
    _zero_vec(zvec, NP // NSUB)
    pltpu.sync_copy(zvec, s_sh.at[pl.ds(sid * (NP // NSUB), NP // NSUB)])
    pltpu.sync_copy(ai_hbm, aitab)
    pltpu.sync_copy(aj_hbm, ajtab)
    plsc.subcore_barrier()

    base0 = wid * EPW

    @pl.loop(0, EPW // KE1)
    def _(b):
        base = base0 + b * KE1
        pltpu.sync_copy(src_hbm.at[pl.ds(base, KE1)], src_v)
        pltpu.sync_copy(dst_hbm.at[pl.ds(base, KE1)], dst_v)
        gather = pltpu.async_copy(x_hbm.at[src_v], rows, sem)

        @pl.loop(0, KE1 // 16)
        def _(g):
            dst16 = dst_v[pl.ds(g * 16, 16)]
            src16 = src_v[pl.ds(g * 16, 16)]
            a16 = _lrelu(plsc.load_gather(aitab, [dst16])
                         + plsc.load_gather(ajtab, [src16]))
            e16 = jnp.exp(jnp.minimum(a16, 60.0))
            ebuf[pl.ds(g * 16, 16)] = e16

        gather.wait()
        _scale_rows(rows, ebuf, KE1)
        pltpu.sync_copy(rows, accsh.at[dst_v], add=True)
        pltpu.sync_copy(ebuf, s_sh.at[dst_v], add=True)

    plsc.subcore_barrier()
    for j in range(8):
        pltpu.sync_copy(accsh.at[pl.ds(sid * 640 + j * 80, 80)], zbuf)
        pltpu.sync_copy(zbuf, acc_out.at[c, pl.ds(sid * 640 + j * 80, 80)])
    pltpu.sync_copy(s_sh.at[pl.ds(sid * 640, 640)], zvec)
    pltpu.sync_copy(zvec, s_out.at[pl.ds(c * NP + sid * 640, 640)])


def _r0_sc(x_hbm, b_hbm, acc_out,
           bidx, rows, zbuf, accsh):
    c = lax.axis_index("c")
    sid = lax.axis_index("s")
    wid = c * NSUB + sid
    _zero_rows(zbuf, 40)
    pltpu.sync_copy(zbuf, accsh.at[pl.ds(sid * 40, 40)])
    plsc.subcore_barrier()
    base = wid * NPW
    pltpu.sync_copy(b_hbm.at[pl.ds(base, NPW)], bidx)
    pltpu.sync_copy(x_hbm.at[pl.ds(base, NPW)], rows)
    pltpu.sync_copy(rows, accsh.at[bidx], add=True)
    plsc.subcore_barrier()
    pltpu.sync_copy(accsh.at[pl.ds(sid * 40, 40)], zbuf)
    pltpu.sync_copy(zbuf, acc_out.at[c, pl.ds(sid * 40, 40)])


def _rt_sc(x_hbm, b_hbm, ob_hbm, xw_hbm,
           acc_out, s_out,
           bidx, rows, xwv, ebuf, obtab,
           zbuf, zvec, accsh, s_sh):
    c = lax.axis_index("c")
    sid = lax.axis_index("s")
    wid = c * NSUB + sid
    _zero_rows(zbuf, 40)
    pltpu.sync_copy(zbuf, accsh.at[pl.ds(sid * 40, 40)])
    _zero_vec(zvec, GP // NSUB)
    pltpu.sync_copy(zvec, s_sh.at[pl.ds(sid * (GP // NSUB), GP // NSUB)])
    pltpu.sync_copy(ob_hbm, obtab)
    plsc.subcore_barrier()
    base = wid * NPW
    pltpu.sync_copy(b_hbm.at[pl.ds(base, NPW)], bidx)
    pltpu.sync_copy(xw_hbm.at[pl.ds(base, NPW)], xwv)
    pltpu.sync_copy(x_hbm.at[pl.ds(base, NPW)], rows)

    @pl.loop(0, NPW // 16)
    def _(g):
        b16 = bidx[pl.ds(g * 16, 16)]
        a16 = _lrelu(plsc.load_gather(obtab, [b16]) + xwv[pl.ds(g * 16, 16)])
        e16 = jnp.exp(jnp.minimum(a16, 60.0))
        ebuf[pl.ds(g * 16, 16)] = e16

    _scale_rows(rows, ebuf, NPW)
    pltpu.sync_copy(rows, accsh.at[bidx], add=True)
    pltpu.sync_copy(ebuf, s_sh.at[bidx], add=True)
    plsc.subcore_barrier()
    pltpu.sync_copy(accsh.at[pl.ds(sid * 40, 40)], zbuf)
    pltpu.sync_copy(zbuf, acc_out.at[c, pl.ds(sid * 40, 40)])
    pltpu.sync_copy(s_sh.at[pl.ds(sid * 40, 40)], zvec)
    pltpu.sync_copy(zvec, s_out.at[pl.ds(c * GP + sid * 40, 40)])


_l0_call = functools.partial(
    pl.kernel, _l0_sc,
    out_type=[jax.ShapeDtypeStruct((NCORES, NP, H), F32),
              jax.ShapeDtypeStruct((NCORES * NP,), F32)],
    mesh=_MESH,
    compiler_params=_SC_PARAMS,
    scratch_types=[
        pltpu.VMEM((KE0,), jnp.int32),
        pltpu.VMEM((KE0,), jnp.int32),
        pltpu.VMEM((KE0, H), F32),
        pltpu.VMEM((KE0, H), F32),
        pltpu.VMEM((KE0,), F32),
        pltpu.VMEM((NP,), F32),
        pltpu.VMEM((H,), F32),
        pltpu.VMEM((80, H), F32),
        pltpu.VMEM((NP // NSUB,), F32),
        pltpu.VMEM_SHARED((NP, H), F32),
        pltpu.VMEM_SHARED((NP,), F32),
        pltpu.SemaphoreType.DMA,
    ],
)

_l1_call = functools.partial(
    pl.kernel, _l1_sc,
    out_type=[jax.ShapeDtypeStruct((NCORES, NP, H), F32),
              jax.ShapeDtypeStruct((NCORES * NP,), F32)],
    mesh=_MESH,
    compiler_params=_SC_PARAMS,
    scratch_types=[
        pltpu.VMEM((KE1,), jnp.int32),
        pltpu.VMEM((KE1,), jnp.int32),
        pltpu.VMEM((KE1, H), F32),
        pltpu.VMEM((KE1,), F32),
        pltpu.VMEM((NP,), F32),
        pltpu.VMEM((NP,), F32),
        pltpu.VMEM((80, H), F32),
        pltpu.VMEM((NP // NSUB,), F32),
        pltpu.VMEM_SHARED((NP, H), F32),
        pltpu.VMEM_SHARED((NP,), F32),
        pltpu.SemaphoreType.DMA,
    ],
)

_r0_call = functools.partial(
    pl.kernel, _r0_sc,
    out_type=jax.ShapeDtypeStruct((NCORES, GP, H), F32),
    mesh=_MESH,
    compiler_params=_SC_PARAMS,
    scratch_types=[
        pltpu.VMEM((NPW,), jnp.int32),
        pltpu.VMEM((NPW, H), F32),
        pltpu.VMEM((40, H), F32),
        pltpu.VMEM_SHARED((GP, H), F32),
    ],
)

_rt_call = functools.partial(
    pl.kernel, _rt_sc,
    out_type=[jax.ShapeDtypeStruct((NCORES, GP, H), F32),
              jax.ShapeDtypeStruct((NCORES * GP,), F32)],
    mesh=_MESH,
    compiler_params=_SC_PARAMS,
    scratch_types=[
        pltpu.VMEM((NPW,), jnp.int32),
        pltpu.VMEM((NPW, H), F32),
        pltpu.VMEM((NPW,), F32),
        pltpu.VMEM((NPW,), F32),
        pltpu.VMEM((GP,), F32),
        pltpu.VMEM((40, H), F32),
        pltpu.VMEM((GP // NSUB,), F32),
        pltpu.VMEM_SHARED((GP, H), F32),
        pltpu.VMEM_SHARED((GP,), F32),
    ],
)


def kernel(raw, edge_index, edge_attr, batch, params):
    p = params
    src = edge_index[0]
    dst = edge_index[1]
    raw_p = jnp.pad(raw, ((0, NP - N), (0, 0)))
    batch_p = jnp.concatenate(
        [batch.astype(jnp.int32), jnp.full((NP - N,), G, jnp.int32)])

    grid_n = NP // NB
    grid_e = E // NB
    row = lambda shape: pl.BlockSpec(shape, lambda i: (i, 0))
    row3 = lambda shape: pl.BlockSpec(shape, lambda i: (0, i, 0))

    wa0i = p['Wa0'][0, :H].reshape(H, 1)
    wa0j = p['Wa0'][0, H:]
    wa1i = p['Wa1'][0, :H].reshape(H, 1)
    wa1j = p['Wa1'][0, H:].reshape(H, 1)
    wami = p['Wam'][0, :H].reshape(H, 1)
    wamj = p['Wam'][0, H:].reshape(H, 1)

    x1, z, ad0 = pl.pallas_call(
        _t1a_body,
        grid=(grid_n,),
        in_specs=[row((NB, F_IN)), _full((H, F_IN)), _full((1, H)),
                  _full((H, F_IN)), _full((1, H)), _full((H, 1)),
                  _full((1, 1))],
        out_specs=[row((NB, H)), row((NB, H)), row((NB, 1))],
        out_shape=[jax.ShapeDtypeStruct((NP, H), F32),
                   jax.ShapeDtypeStruct((NP, H), F32),
                   jax.ShapeDtypeStruct((NP, 1), F32)],
    )(raw_p, p['W1'], p['b1'].reshape(1, H),
      p['Wn'][:, :F_IN], p['bn'].reshape(1, H), wa0i,
      p['ba0'].reshape(1, 1))
    ad0 = ad0.reshape(NP)

    C = pl.pallas_call(
        _t1b_body,
        grid=(grid_e,),
        in_specs=[row((NB, DE)), _full((H, DE))],
        out_specs=row((NB, H)),
        out_shape=jax.ShapeDtypeStruct((E, H), F32),
    )(edge_attr, p['Wn'][:, F_IN:])

    acc0, s0 = _l0_call()(z, C, src, dst, ad0, wa0j)

    def node_update(acc, s, xp, tag, wi, wj, bia):
        return pl.pallas_call(
            _node_update_body,
            grid=(grid_n,),
            in_specs=[row3((NCORES, NB, H)), row3((NCORES, NB, 1)),
                      row((NB, H)), _full((H, H)), _full((1, H)),
                      _full((3 * H, H)), _full((3 * H, H)),
                      _full((1, 3 * H)), _full((1, 3 * H)),
                      _full((H, 1)), _full((H, 1)), _full((1, 1))],
            out_specs=[row((NB, H)), row((NB, 1)), row((NB, 1))],
            out_shape=[jax.ShapeDtypeStruct((NP, H), F32),
                       jax.ShapeDtypeStruct((NP, 1), F32),
                       jax.ShapeDtypeStruct((NP, 1), F32)],
        )(acc, s.reshape(NCORES, NP, 1), xp,
          p['Wat' + tag], p['bat' + tag].reshape(1, H),
          p['Wih' + tag], p['Whh' + tag],
          p['bih' + tag].reshape(1, 3 * H), p['bhh' + tag].reshape(1, 3 * H),
          wi, wj, bia)

    x2, ai1, aj1 = node_update(acc0, s0, x1, '0', wa1i, wa1j,
                               p['ba1'].reshape(1, 1))

    acc1, s1 = _l1_call()(x2, src, dst, ai1.reshape(NP), aj1.reshape(NP))

    x3, xw, _ = node_update(acc1, s1, x2, '1', wamj,
                            jnp.zeros((H, 1), F32), p['bam'].reshape(1, 1))
    xw = xw.reshape(NP)

    accr = _r0_call()(x3, batch_p)
    out0, ob0 = pl.pallas_call(
        _t4_body,
        grid=(1,),
        in_specs=[row3((NCORES, G, H)), _full((H, 1))],
        out_specs=[row((G, H)), row((G, 1))],
        out_shape=[jax.ShapeDtypeStruct((G, H), F32),
                   jax.ShapeDtypeStruct((G, 1), F32)],
    )(accr, wami)

    out, ob = out0, ob0
    for t in range(NUM_TIMESTEPS):
        ob_pad = jnp.pad(ob.reshape(G), (0, GP - G))
        accm, sm = _rt_call()(x3, batch_p, ob_pad, xw)
        last = t == NUM_TIMESTEPS - 1
        if not last:
            out, ob = pl.pallas_call(
                _readout_update_body,
                grid=(1,),
                in_specs=[row3((NCORES, G, H)), row3((NCORES, G, 1)),
                          row((G, H)), _full((H, H)), _full((1, H)),
                          _full((3 * H, H)), _full((3 * H, H)),
                          _full((1, 3 * H)), _full((1, 3 * H)),
                          _full((H, 1))],
                out_specs=[row((G, H)), row((G, 1))],
                out_shape=[jax.ShapeDtypeStruct((G, H), F32),
                           jax.ShapeDtypeStruct((G, 1), F32)],
            )(accm, sm.reshape(NCORES, GP, 1), out,
              p['Watm'], p['batm'].reshape(1, H),
              p['Wihm'], p['Whhm'],
              p['bihm'].reshape(1, 3 * H), p['bhhm'].reshape(1, 3 * H),
              wami)
        else:
            return pl.pallas_call(
                _final_body,
                grid=(1,),
                in_specs=[row3((NCORES, G, H)), row3((NCORES, G, 1)),
                          row((G, H)), _full((H, H)), _full((1, H)),
                          _full((3 * H, H)), _full((3 * H, H)),
                          _full((1, 3 * H)), _full((1, 3 * H)),
                          _full((OUT, H)), _full((1, OUT))],
                out_specs=row((G, OUT)),
                out_shape=jax.ShapeDtypeStruct((G, OUT), F32),
            )(accm, sm.reshape(NCORES, GP, 1), out,
              p['Watm'], p['batm'].reshape(1, H),
              p['Wihm'], p['Whhm'],
              p['bihm'].reshape(1, 3 * H), p['bhhm'].reshape(1, 3 * H),
              p['W2'], p['b2'].reshape(1, OUT))


# speedup vs baseline: 7.2497x; 1.1033x over previous
import dataclasses
import functools

import jax
import jax.numpy as jnp
from jax import lax
from jax.experimental import pallas as pl
from jax.experimental.pallas import tpu as pltpu
from jax.experimental.pallas import tpu_sc as plsc

N = 10000
E = 320000
F_IN = 128
H = 128
DE = 16
OUT = 128
G = 512
NUM_TIMESTEPS = 2

NP = 10240
GP = 640
NB = 512
NCORES = 2
NSUB = 16
NW = NCORES * NSUB
EPW = E // NW
EP = 327680
KE = 64
EPWP = EP // NW
NBLK = EPWP // KE
NPW = NP // NW
F32 = jnp.float32

_MESH = plsc.VectorSubcoreMesh(core_axis_name="c", subcore_axis_name="s",
                               num_cores=NCORES, num_subcores=NSUB)

_SC_PARAMS = pltpu.CompilerParams()
if "needs_layout_passes" in pltpu.CompilerParams.__dataclass_fields__:
    _SC_PARAMS = dataclasses.replace(_SC_PARAMS, needs_layout_passes=False)


def _lrelu(v):
    return jnp.where(v > 0, v, 0.01 * v)


def _elu(v):
    return jnp.where(v > 0, v, jnp.exp(jnp.minimum(v, 0.0)) - 1.0)


def _dot_t(a, b):
    return lax.dot_general(a, b, (((1,), (1,)), ((), ())),
                           preferred_element_type=F32)


def _t1a_body(raw_ref, w1_ref, b1_ref, wn1_ref, bn_ref, wa0i_ref, ba0_ref,
              x1_ref, z_ref, ad_ref):
    raw = raw_ref[...]
    x1 = _lrelu(_dot_t(raw, w1_ref[...]) + b1_ref[...])
    x1_ref[...] = x1
    z_ref[...] = _dot_t(raw, wn1_ref[...]) + bn_ref[...]
    ad_ref[...] = jnp.dot(x1, wa0i_ref[...], preferred_element_type=F32) \
        + ba0_ref[...]


def _t1b_body(ea_ref, wn2_ref, c_ref):
    c_ref[...] = _dot_t(ea_ref[...], wn2_ref[...])


def _gru(xg, hg, wih, whh, bih, bhh):
    gi = _dot_t(xg, wih) + bih
    gh = _dot_t(hg, whh) + bhh
    r = jax.nn.sigmoid(gi[:, :H] + gh[:, :H])
    z = jax.nn.sigmoid(gi[:, H:2 * H] + gh[:, H:2 * H])
    n = jnp.tanh(gi[:, 2 * H:] + r * gh[:, 2 * H:])
    return (1.0 - z) * n + z * hg


def _node_update_body(acc_ref, s_ref, xp_ref, wat_ref, bat_ref,
                      wih_ref, whh_ref, bih_ref, bhh_ref,
                      wi_ref, wj_ref, bia_ref,
                      x_ref, ai_ref, aj_ref):
    ssum = s_ref[0] + s_ref[1]
    inv = 1.0 / (ssum + 1e-16)
    sa = ssum * inv
    hpre = (acc_ref[0] + acc_ref[1]) * inv
    h = _elu(_dot_t(hpre, wat_ref[...]) + bat_ref[...] * sa)
    xnew = jax.nn.relu(_gru(h, xp_ref[...], wih_ref[...], whh_ref[...],
                            bih_ref[...], bhh_ref[...]))
    x_ref[...] = xnew
    ai_ref[...] = jnp.dot(xnew, wi_ref[...], preferred_element_type=F32) \
        + bia_ref[...]
    aj_ref[...] = jnp.dot(xnew, wj_ref[...], preferred_element_type=F32)


def _t4_body(accr_ref, wami_ref, out_ref, ob_ref):
    out0 = jax.nn.relu(accr_ref[0] + accr_ref[1])
    out_ref[...] = out0
    ob_ref[...] = jnp.dot(out0, wami_ref[...], preferred_element_type=F32)


def _readout_update_body(acc_ref, s_ref, op_ref, wat_ref, bat_ref,
                         wih_ref, whh_ref, bih_ref, bhh_ref, wami_ref,
                         out_ref, ob_ref):
    ssum = s_ref[0] + s_ref[1]
    inv = 1.0 / (ssum + 1e-16)
    sa = ssum * inv
    hpre = (acc_ref[0] + acc_ref[1]) * inv
    h = _elu(_dot_t(hpre, wat_ref[...]) + bat_ref[...] * sa)
    onew = jax.nn.relu(_gru(h, op_ref[...], wih_ref[...], whh_ref[...],
                            bih_ref[...], bhh_ref[...]))
    out_ref[...] = onew
    ob_ref[...] = jnp.dot(onew, wami_ref[...], preferred_element_type=F32)


def _final_body(acc_ref, s_ref, op_ref, wat_ref, bat_ref,
                wih_ref, whh_ref, bih_ref, bhh_ref, w2_ref, b2_ref,
                out_ref):
    ssum = s_ref[0] + s_ref[1]
    inv = 1.0 / (ssum + 1e-16)
    sa = ssum * inv
    hpre = (acc_ref[0] + acc_ref[1]) * inv
    h = _elu(_dot_t(hpre, wat_ref[...]) + bat_ref[...] * sa)
    onew = jax.nn.relu(_gru(h, op_ref[...], wih_ref[...], whh_ref[...],
                            bih_ref[...], bhh_ref[...]))
    out_ref[...] = _dot_t(onew, w2_ref[...]) + b2_ref[...]


def _full(shape):
    return pl.BlockSpec(shape, lambda i: tuple(0 for _ in shape))


def _zero_vec(ref, n):
    @pl.loop(0, n, step=16)
    def _(i):
        ref[pl.ds(i, 16)] = jnp.zeros((16,), F32)


def _zero_rows(ref, rows):
    @pl.loop(0, rows)
    def _(i):
        for k in range(8):
            ref[i, pl.ds(k * 16, 16)] = jnp.zeros((16,), F32)


def _scale_rows(rows_ref, ebuf, nrows):
    @pl.loop(0, nrows)
    def _(i):
        eb = plsc.load_gather(ebuf, [jnp.full((16,), i, jnp.int32)])
        for k in range(8):
            rows_ref[i, pl.ds(k * 16, 16)] = rows_ref[i, pl.ds(k * 16, 16)] * eb


def _l0_sc(z_hbm, c_hbm, src_hbm, dst_hbm, ad_hbm, wa_hbm,
           acc_out, s_out,
           src0, src1, dst0, dst1, zr0, zr1, cr0, cr1, ebuf,
           adtab, watab, zvec, accsh, s_sh,
           sis0, sis1, sid0, sid1, sg0, sg1, sc0, sc1):
    c = lax.axis_index("c")
    sid = lax.axis_index("s")
    wid = c * NSUB + sid
    _zero_rows(cr0, KE)
    for j in range(640 // KE):
        pltpu.sync_copy(cr0, accsh.at[pl.ds(sid * 640 + j * KE, KE)])
    _zero_vec(zvec, NP // NSUB)
    pltpu.sync_copy(zvec, s_sh.at[pl.ds(sid * (NP // NSUB), NP // NSUB)])
    pltpu.sync_copy(ad_hbm, adtab)
    pltpu.sync_copy(wa_hbm, watab)
    plsc.subcore_barrier()

    base0 = wid * EPWP

    def start_idx(b, sref, dref, ss, sd):
        bb = base0 + jnp.minimum(b, NBLK - 1) * KE
        pltpu.async_copy(src_hbm.at[pl.ds(bb, KE)], sref, ss)
        pltpu.async_copy(dst_hbm.at[pl.ds(bb, KE)], dref, sd)

    def wait_idx(sref, dref, ss, sd):
        pltpu.make_async_copy(src_hbm.at[pl.ds(base0, KE)], sref, ss).wait()
        pltpu.make_async_copy(dst_hbm.at[pl.ds(base0, KE)], dref, sd).wait()

    def start_rows(b, sref, zr, cr, sg, sc):
        bb = base0 + jnp.minimum(b, NBLK - 1) * KE
        pltpu.async_copy(z_hbm.at[sref], zr, sg)
        pltpu.async_copy(c_hbm.at[pl.ds(bb, KE)], cr, sc)

    def wait_rows(sref, zr, cr, sg, sc):
        pltpu.make_async_copy(z_hbm.at[sref], zr, sg).wait()
        pltpu.make_async_copy(c_hbm.at[pl.ds(base0, KE)], cr, sc).wait()

    def compute(dref, zr, cr):
        @pl.loop(0, KE // 16)
        def _(g):
            dst16 = dref[pl.ds(g * 16, 16)]
            ad16 = plsc.load_gather(adtab, [dst16])

            def edge_dot(e2, qv):
                i = g * 16 + e2
                q = jnp.zeros((16,), F32)
                for k in range(8):
                    xs = zr[i, pl.ds(k * 16, 16)] \
                        + cr[i, pl.ds(k * 16, 16)]
                    xs = jnp.where(xs > 0, xs, 0.01 * xs)
                    cr[i, pl.ds(k * 16, 16)] = xs
                    q = q + xs * watab[pl.ds(k * 16, 16)]
                return jnp.where(lax.iota(jnp.int32, 16) == e2,
                                 jnp.sum(q), qv)

            qv = lax.fori_loop(0, 16, edge_dot, jnp.zeros((16,), F32))
            a16 = _lrelu(ad16 + qv)
            e16 = jnp.exp(jnp.minimum(a16, 60.0))
            ebuf[pl.ds(g * 16, 16)] = e16

        _scale_rows(cr, ebuf, KE)
        pltpu.sync_copy(cr, accsh.at[dref], add=True)
        pltpu.sync_copy(ebuf, s_sh.at[dref], add=True)

    start_idx(0, src0, dst0, sis0, sid0)
    wait_idx(src0, dst0, sis0, sid0)
    start_rows(0, src0, zr0, cr0, sg0, sc0)
    start_idx(1, src1, dst1, sis1, sid1)

    @pl.loop(0, NBLK // 2)
    def _(i):
        b = i * 2
        wait_idx(src1, dst1, sis1, sid1)
        start_rows(b + 1, src1, zr1, cr1, sg1, sc1)
        wait_rows(src0, zr0, cr0, sg0, sc0)
        compute(dst0, zr0, cr0)
        start_idx(b + 2, src0, dst0, sis0, sid0)

        wait_idx(src0, dst0, sis0, sid0)
        start_rows(b + 2, src0, zr0, cr0, sg0, sc0)
        wait_rows(src1, zr1, cr1, sg1, sc1)
        compute(dst1, zr1, cr1)
        start_idx(b + 3, src1, dst1, sis1, sid1)

    wait_idx(src1, dst1, sis1, sid1)
    wait_rows(src0, zr0, cr0, sg0, sc0)

    plsc.subcore_barrier()
    for j in range(640 // KE):
        pltpu.sync_copy(accsh.at[pl.ds(sid * 640 + j * KE, KE)], cr0)
        pltpu.sync_copy(cr0, acc_out.at[c, pl.ds(sid * 640 + j * KE, KE)])
    pltpu.sync_copy(s_sh.at[pl.ds(sid * 640, 640)], zvec)
    pltpu.sync_copy(zvec, s_out.at[pl.ds(c * NP + sid * 640, 640)])


def _l1_sc(x_hbm, src_hbm, dst_hbm, ai_hbm, aj_hbm,
           acc_out, s_out,
           src0, src1, dst0, dst1, r0, r1, ebuf,
           aitab, ajtab, zvec, accsh, s_sh,
           sis0, sis1, sid0, sid1, sg0, sg1):
    c = lax.axis_index("c")
    sid = lax.axis_index("s")
    wid = c * NSUB + sid
    _zero_rows(r0, KE)
    for j in range(640 // KE):
        pltpu.sync_copy(r0, accsh.at[pl.ds(sid * 640 + j * KE, KE)])
    _zero_vec(zvec, NP // NSUB)
    pltpu.sync_copy(zvec, s_sh.at[pl.ds(sid * (NP // NSUB), NP // NSUB)])
    pltpu.sync_copy(ai_hbm, aitab)
    pltpu.sync_copy(aj_hbm, ajtab)
    plsc.subcore_barrier()

    base0 = wid * EPWP

    def start_idx(b, sref, dref, ss, sd):
        bb = base0 + jnp.minimum(b, NBLK - 1) * KE
        pltpu.async_copy(src_hbm.at[pl.ds(bb, KE)], sref, ss)
        pltpu.async_copy(dst_hbm.at[pl.ds(bb, KE)], dref, sd)

    def wait_idx(sref, dref, ss, sd):
        pltpu.make_async_copy(src_hbm.at[pl.ds(base0, KE)], sref, ss).wait()
        pltpu.make_async_copy(dst_hbm.at[pl.ds(base0, KE)], dref, sd).wait()

    def start_rows(sref, rr, sg):
        pltpu.async_copy(x_hbm.at[sref], rr, sg)

    def wait_rows(sref, rr, sg):
        pltpu.make_async_copy(x_hbm.at[sref], rr, sg).wait()

    def compute(sref, dref, rr):
        @pl.loop(0, KE // 16)
        def _(g):
            dst16 = dref[pl.ds(g * 16, 16)]
            src16 = sref[pl.ds(g * 16, 16)]
            a16 = _lrelu(plsc.load_gather(aitab, [dst16])
                         + plsc.load_gather(ajtab, [src16]))
            e16 = jnp.exp(jnp.minimum(a16, 60.0))
            ebuf[pl.ds(g * 16, 16)] = e16

        _scale_rows(rr, ebuf, KE)
        pltpu.sync_copy(rr, accsh.at[dref], add=True)
        pltpu.sync_copy(ebuf, s_sh.at[dref], add=True)

    start_idx(0, src0, dst0, sis0, sid0)
    wait_idx(src0, dst0, sis0, sid0)
    start_rows(src0, r0, sg0)
    start_idx(1, src1, dst1, sis1, sid1)

    @pl.loop(0, NBLK // 2)
    def _(i):
        b = i * 2
        wait_idx(src1, dst1, sis1, sid1)
        start_rows(src1, r1, sg1)
        wait_rows(src0, r0, sg0)
        compute(src0, dst0, r0)
        start_idx(b + 2, src0, dst0, sis0, sid0)

        wait_idx(src0, dst0, sis0, sid0)
        start_rows(src0, r0, sg0)
        wait_rows(src1, r1, sg1)
        compute(src1, dst1, r1)
        start_idx(b + 3, src1, dst1, sis1, sid1)

    wait_idx(src1, dst1, sis1, sid1)
    wait_rows(src0, r0, sg0)

    plsc.subcore_barrier()
    for j in range(640 // KE):
        pltpu.sync_copy(accsh.at[pl.ds(sid * 640 + j * KE, KE)], r0)
        pltpu.sync_copy(r0, acc_out.at[c, pl.ds(sid * 640 + j * KE, KE)])
    pltpu.sync_copy(s_sh.at[pl.ds(sid * 640, 640)], zvec)
    pltpu.sync_copy(zvec, s_out.at[pl.ds(c * NP + sid * 640, 640)])


def _r0_sc(x_hbm, b_hbm, acc_out,
           bidx, rows, zbuf, accsh):
    c = lax.axis_index("c")
    sid = lax.axis_index("s")
    wid = c * NSUB + sid
    _zero_rows(zbuf, 40)
    pltpu.sync_copy(zbuf, accsh.at[pl.ds(sid * 40, 40)])
    plsc.subcore_barrier()
    base = wid * NPW
    pltpu.sync_copy(b_hbm.at[pl.ds(base, NPW)], bidx)
    pltpu.sync_copy(x_hbm.at[pl.ds(base, NPW)], rows)
    pltpu.sync_copy(rows, accsh.at[bidx], add=True)
    plsc.subcore_barrier()
    pltpu.sync_copy(accsh.at[pl.ds(sid * 40, 40)], zbuf)
    pltpu.sync_copy(zbuf, acc_out.at[c, pl.ds(sid * 40, 40)])


def _rt_sc(x_hbm, b_hbm, ob_hbm, xw_hbm,
           acc_out, s_out,
           bidx, rows, xwv, ebuf, obtab,
           zbuf, zvec, accsh, s_sh):
    c = lax.axis_index("c")
    sid = lax.axis_index("s")
    wid = c * NSUB + sid
    _zero_rows(zbuf, 40)
    pltpu.sync_copy(zbuf, accsh.at[pl.ds(sid * 40, 40)])
    _zero_vec(zvec, GP // NSUB)
    pltpu.sync_copy(zvec, s_sh.at[pl.ds(sid * (GP // NSUB), GP // NSUB)])
    pltpu.sync_copy(ob_hbm, obtab)
    plsc.subcore_barrier()
    base = wid * NPW
    pltpu.sync_copy(b_hbm.at[pl.ds(base, NPW)], bidx)
    pltpu.sync_copy(xw_hbm.at[pl.ds(base, NPW)], xwv)
    pltpu.sync_copy(x_hbm.at[pl.ds(base, NPW)], rows)

    @pl.loop(0, NPW // 16)
    def _(g):
        b16 = bidx[pl.ds(g * 16, 16)]
        a16 = _lrelu(plsc.load_gather(obtab, [b16]) + xwv[pl.ds(g * 16, 16)])
        e16 = jnp.exp(jnp.minimum(a16, 60.0))
        ebuf[pl.ds(g * 16, 16)] = e16

    _scale_rows(rows, ebuf, NPW)
    pltpu.sync_copy(rows, accsh.at[bidx], add=True)
    pltpu.sync_copy(ebuf, s_sh.at[bidx], add=True)
    plsc.subcore_barrier()
    pltpu.sync_copy(accsh.at[pl.ds(sid * 40, 40)], zbuf)
    pltpu.sync_copy(zbuf, acc_out.at[c, pl.ds(sid * 40, 40)])
    pltpu.sync_copy(s_sh.at[pl.ds(sid * 40, 40)], zvec)
    pltpu.sync_copy(zvec, s_out.at[pl.ds(c * GP + sid * 40, 40)])


_l0_call = functools.partial(
    pl.kernel, _l0_sc,
    out_type=[jax.ShapeDtypeStruct((NCORES, NP, H), F32),
              jax.ShapeDtypeStruct((NCORES * NP,), F32)],
    mesh=_MESH,
    compiler_params=_SC_PARAMS,
    scratch_types=[
        pltpu.VMEM((KE,), jnp.int32),
        pltpu.VMEM((KE,), jnp.int32),
        pltpu.VMEM((KE,), jnp.int32),
        pltpu.VMEM((KE,), jnp.int32),
        pltpu.VMEM((KE, H), F32),
        pltpu.VMEM((KE, H), F32),
        pltpu.VMEM((KE, H), F32),
        pltpu.VMEM((KE, H), F32),
        pltpu.VMEM((KE,), F32),
        pltpu.VMEM((NP,), F32),
        pltpu.VMEM((H,), F32),
        pltpu.VMEM((NP // NSUB,), F32),
        pltpu.VMEM_SHARED((NP, H), F32),
        pltpu.VMEM_SHARED((NP,), F32),
    ] + [pltpu.SemaphoreType.DMA] * 8,
)

_l1_call = functools.partial(
    pl.kernel, _l1_sc,
    out_type=[jax.ShapeDtypeStruct((NCORES, NP, H), F32),
              jax.ShapeDtypeStruct((NCORES * NP,), F32)],
    mesh=_MESH,
    compiler_params=_SC_PARAMS,
    scratch_types=[
        pltpu.VMEM((KE,), jnp.int32),
        pltpu.VMEM((KE,), jnp.int32),
        pltpu.VMEM((KE,), jnp.int32),
        pltpu.VMEM((KE,), jnp.int32),
        pltpu.VMEM((KE, H), F32),
        pltpu.VMEM((KE, H), F32),
        pltpu.VMEM((KE,), F32),
        pltpu.VMEM((NP,), F32),
        pltpu.VMEM((NP,), F32),
        pltpu.VMEM((NP // NSUB,), F32),
        pltpu.VMEM_SHARED((NP, H), F32),
        pltpu.VMEM_SHARED((NP,), F32),
    ] + [pltpu.SemaphoreType.DMA] * 6,
)

_r0_call = functools.partial(
    pl.kernel, _r0_sc,
    out_type=jax.ShapeDtypeStruct((NCORES, GP, H), F32),
    mesh=_MESH,
    compiler_params=_SC_PARAMS,
    scratch_types=[
        pltpu.VMEM((NPW,), jnp.int32),
        pltpu.VMEM((NPW, H), F32),
        pltpu.VMEM((40, H), F32),
        pltpu.VMEM_SHARED((GP, H), F32),
    ],
)

_rt_call = functools.partial(
    pl.kernel, _rt_sc,
    out_type=[jax.ShapeDtypeStruct((NCORES, GP, H), F32),
              jax.ShapeDtypeStruct((NCORES * GP,), F32)],
    mesh=_MESH,
    compiler_params=_SC_PARAMS,
    scratch_types=[
        pltpu.VMEM((NPW,), jnp.int32),
        pltpu.VMEM((NPW, H), F32),
        pltpu.VMEM((NPW,), F32),
        pltpu.VMEM((NPW,), F32),
        pltpu.VMEM((GP,), F32),
        pltpu.VMEM((40, H), F32),
        pltpu.VMEM((GP // NSUB,), F32),
        pltpu.VMEM_SHARED((GP, H), F32),
        pltpu.VMEM_SHARED((GP,), F32),
    ],
)


def kernel(raw, edge_index, edge_attr, batch, params):
    p = params
    src = jnp.pad(edge_index[0], (0, EP - E))
    dst = jnp.pad(edge_index[1], (0, EP - E), constant_values=N)
    ea_p = jnp.pad(edge_attr, ((0, EP - E), (0, 0)))
    raw_p = jnp.pad(raw, ((0, NP - N), (0, 0)))
    batch_p = jnp.concatenate(
        [batch.astype(jnp.int32), jnp.full((NP - N,), G, jnp.int32)])

    grid_n = NP // NB
    grid_e = EP // NB
    row = lambda shape: pl.BlockSpec(shape, lambda i: (i, 0))
    row3 = lambda shape: pl.BlockSpec(shape, lambda i: (0, i, 0))

    wa0i = p['Wa0'][0, :H].reshape(H, 1)
    wa0j = p['Wa0'][0, H:]
    wa1i = p['Wa1'][0, :H].reshape(H, 1)
    wa1j = p['Wa1'][0, H:].reshape(H, 1)
    wami = p['Wam'][0, :H].reshape(H, 1)
    wamj = p['Wam'][0, H:].reshape(H, 1)

    x1, z, ad0 = pl.pallas_call(
        _t1a_body,
        grid=(grid_n,),
        in_specs=[row((NB, F_IN)), _full((H, F_IN)), _full((1, H)),
                  _full((H, F_IN)), _full((1, H)), _full((H, 1)),
                  _full((1, 1))],
        out_specs=[row((NB, H)), row((NB, H)), row((NB, 1))],
        out_shape=[jax.ShapeDtypeStruct((NP, H), F32),
                   jax.ShapeDtypeStruct((NP, H), F32),
                   jax.ShapeDtypeStruct((NP, 1), F32)],
    )(raw_p, p['W1'], p['b1'].reshape(1, H),
      p['Wn'][:, :F_IN], p['bn'].reshape(1, H), wa0i,
      p['ba0'].reshape(1, 1))
    ad0 = ad0.reshape(NP)

    C = pl.pallas_call(
        _t1b_body,
        grid=(grid_e,),
        in_specs=[row((NB, DE)), _full((H, DE))],
        out_specs=row((NB, H)),
        out_shape=jax.ShapeDtypeStruct((EP, H), F32),
    )(ea_p, p['Wn'][:, F_IN:])

    acc0, s0 = _l0_call()(z, C, src, dst, ad0, wa0j)

    def node_update(acc, s, xp, tag, wi, wj, bia):
        return pl.pallas_call(
            _node_update_body,
            grid=(grid_n,),
            in_specs=[row3((NCORES, NB, H)), row3((NCORES, NB, 1)),
                      row((NB, H)), _full((H, H)), _full((1, H)),
                      _full((3 * H, H)), _full((3 * H, H)),
                      _full((1, 3 * H)), _full((1, 3 * H)),
                      _full((H, 1)), _full((H, 1)), _full((1, 1))],
            out_specs=[row((NB, H)), row((NB, 1)), row((NB, 1))],
            out_shape=[jax.ShapeDtypeStruct((NP, H), F32),
                       jax.ShapeDtypeStruct((NP, 1), F32),
                       jax.ShapeDtypeStruct((NP, 1), F32)],
        )(acc, s.reshape(NCORES, NP, 1), xp,
          p['Wat' + tag], p['bat' + tag].reshape(1, H),
          p['Wih' + tag], p['Whh' + tag],
          p['bih' + tag].reshape(1, 3 * H), p['bhh' + tag].reshape(1, 3 * H),
          wi, wj, bia)

    x2, ai1, aj1 = node_update(acc0, s0, x1, '0', wa1i, wa1j,
                               p['ba1'].reshape(1, 1))

    acc1, s1 = _l1_call()(x2, src, dst, ai1.reshape(NP), aj1.reshape(NP))

    x3, xw, _ = node_update(acc1, s1, x2, '1', wamj,
                            jnp.zeros((H, 1), F32), p['bam'].reshape(1, 1))
    xw = xw.reshape(NP)

    accr = _r0_call()(x3, batch_p)
    out0, ob0 = pl.pallas_call(
        _t4_body,
        grid=(1,),
        in_specs=[row3((NCORES, G, H)), _full((H, 1))],
        out_specs=[row((G, H)), row((G, 1))],
        out_shape=[jax.ShapeDtypeStruct((G, H), F32),
                   jax.ShapeDtypeStruct((G, 1), F32)],
    )(accr, wami)

    out, ob = out0, ob0
    for t in range(NUM_TIMESTEPS):
        ob_pad = jnp.pad(ob.reshape(G), (0, GP - G))
        accm, sm = _rt_call()(x3, batch_p, ob_pad, xw)
        last = t == NUM_TIMESTEPS - 1
        if not last:
            out, ob = pl.pallas_call(
                _readout_update_body,
                grid=(1,),
                in_specs=[row3((NCORES, G, H)), row3((NCORES, G, 1)),
                          row((G, H)), _full((H, H)), _full((1, H)),
                          _full((3 * H, H)), _full((3 * H, H)),
                          _full((1, 3 * H)), _full((1, 3 * H)),
                          _full((H, 1))],
                out_specs=[row((G, H)), row((G, 1))],
                out_shape=[jax.ShapeDtypeStruct((G, H), F32),
                           jax.ShapeDtypeStruct((G, 1), F32)],
            )(accm, sm.reshape(NCORES, GP, 1), out,
              p['Watm'], p['batm'].reshape(1, H),
              p['Wihm'], p['Whhm'],
              p['bihm'].reshape(1, 3 * H), p['bhhm'].reshape(1, 3 * H),
              wami)
        else:
            return pl.pallas_call(
                _final_body,
                grid=(1,),
                in_specs=[row3((NCORES, G, H)), row3((NCORES, G, 1)),
                          row((G, H)), _full((H, H)), _full((1, H)),
                          _full((3 * H, H)), _full((3 * H, H)),
                          _full((1, 3 * H)), _full((1, 3 * H)),
                          _full((OUT, H)), _full((1, OUT))],
                out_specs=row((G, OUT)),
                out_shape=jax.ShapeDtypeStruct((G, OUT), F32),
            )(accm, sm.reshape(NCORES, GP, 1), out,
              p['Watm'], p['batm'].reshape(1, H),
              p['Wihm'], p['Whhm'],
              p['bihm'].reshape(1, 3 * H), p['bhhm'].reshape(1, 3 * H),
              p['W2'], p['b2'].reshape(1, OUT))
